# Initial kernel scaffold; baseline (speedup 1.0000x reference)
import jax
import jax.numpy as jnp
from jax.experimental import pallas as pl


def kernel(features, indices, saved_features, saved_rks):
    raise NotImplementedError("write your pallas kernel here")


# speedup vs baseline: 3.6880x; 3.6880x over previous
import functools

import jax
import jax.numpy as jnp
from jax import lax
from jax.experimental import pallas as pl
from jax.experimental.pallas import tpu as pltpu
from jax.experimental.pallas import tpu_sc as plsc

_TEMP = 0.07
_TOP_K = 15
_RKS_COLS = 50

_NC = 2
_NS = 16
_NW = _NC * _NS
_ROWS_PER_W = 64
_PAD_ROWS = _NW * _ROWS_PER_W


def _sc_gather_body(indices_hbm, rks_hbm, feat_hbm, out_hbm,
                    idx_v, fidx_v, neigh_v, rows_v, sem):
    wid = lax.axis_index("s") * _NC + lax.axis_index("c")
    base = wid * _ROWS_PER_W
    k = wid // 2
    i0 = (wid % 2) * _ROWS_PER_W
    pltpu.sync_copy(indices_hbm.at[pl.ds(i0, _ROWS_PER_W)], idx_v)
    for ch in range(_ROWS_PER_W // 16):
        gi = idx_v[pl.ds(ch * 16, 16)]
        fidx_v[pl.ds(ch * 16, 16)] = gi * _RKS_COLS + k
    pltpu.async_copy(rks_hbm.at[fidx_v], neigh_v, sem).wait()
    pltpu.async_copy(feat_hbm.at[neigh_v], rows_v, sem).wait()
    pltpu.sync_copy(rows_v, out_hbm.at[pl.ds(base, _ROWS_PER_W)])


def _sc_gather(indices, rks_flat, saved_features):
    mesh = plsc.VectorSubcoreMesh(core_axis_name="c", subcore_axis_name="s")
    return pl.kernel(
        _sc_gather_body,
        out_type=jax.ShapeDtypeStruct((_PAD_ROWS, 128), jnp.float32),
        mesh=mesh,
        scratch_types=[
            pltpu.VMEM((_ROWS_PER_W,), jnp.int32),
            pltpu.VMEM((_ROWS_PER_W,), jnp.int32),
            pltpu.VMEM((_ROWS_PER_W,), jnp.int32),
            pltpu.VMEM((_ROWS_PER_W, 128), jnp.float32),
            pltpu.SemaphoreType.DMA,
        ],
    )(indices, rks_flat, saved_features)


def _sim_from(d2):
    d = jnp.sqrt(jnp.maximum(d2, 0.0))
    return (1.0 + 1.0 / (1.0 + d)) / _TEMP


def _dotT(x, y):
    return lax.dot_general(x, y, (((1,), (1,)), ((), ())),
                           preferred_element_type=jnp.float32,
                           precision=lax.Precision.HIGHEST)


def _tc_loss_body(a_ref, g_ref, out_ref):
    A = a_ref[...]
    a2 = jnp.sum(A * A, axis=1)

    paa = _dotT(A, A)
    d2aa = a2[:, None] + a2[None, :] - 2.0 * paa
    orig = _sim_from(d2aa)

    S = jnp.zeros((256, 128), jnp.float32)
    ST = jnp.zeros((128, 256), jnp.float32)
    for k in range(_TOP_K):
        Gk = g_ref[pl.ds(k * 128, 128), :]
        g2 = jnp.sum(Gk * Gk, axis=1)
        p = _dotT(A, Gk)
        S = S + _sim_from(a2[:, None] + g2[None, :] - 2.0 * p)
        pT = _dotT(Gk, A)
        ST = ST + _sim_from(g2[:, None] + a2[None, :] - 2.0 * pT)

    Sn = S * (1.0 / _TOP_K)
    SnT = ST * (1.0 / _TOP_K)
    acc = jnp.concatenate([Sn, Sn], axis=1)
    acc_sym = jnp.concatenate([SnT, SnT], axis=0)

    adc = jnp.sqrt(acc * acc + acc_sym * acc_sym + orig * orig)

    rowmax = jnp.max(adc, axis=1)[:, None]
    logits = adc - rowmax
    ii = lax.broadcasted_iota(jnp.int32, (256, 256), 0)
    jj = lax.broadcasted_iota(jnp.int32, (256, 256), 1)
    offdiag = ii != jj
    pos = jnp.abs(ii - jj) == 128
    denom = jnp.sum(jnp.where(offdiag, jnp.exp(logits), 0.0), axis=1)
    posval = jnp.sum(jnp.where(pos, logits, 0.0), axis=1)
    loss_i = jnp.log(denom) - posval
    out_ref[0, 0] = jnp.mean(loss_i)


def _tc_loss(A, G):
    return pl.pallas_call(
        _tc_loss_body,
        out_shape=jax.ShapeDtypeStruct((1, 1), jnp.float32),
        in_specs=[
            pl.BlockSpec(memory_space=pltpu.VMEM),
            pl.BlockSpec(memory_space=pltpu.VMEM),
        ],
        out_specs=pl.BlockSpec(memory_space=pltpu.SMEM),
    )(A, G)


def kernel(features, indices, saved_features, saved_rks):
    A = jnp.concatenate([features[:, 0, :], features[:, 1, :]], axis=0)
    rks_flat = saved_rks.reshape(-1)
    G = _sc_gather(indices, rks_flat, saved_features)
    out = _tc_loss(A, G)
    return out[0, 0]


# speedup vs baseline: 7.0345x; 1.9074x over previous
import functools

import jax
import jax.numpy as jnp
from jax import lax
from jax.experimental import pallas as pl
from jax.experimental.pallas import tpu as pltpu
from jax.experimental.pallas import tpu_sc as plsc

_TEMP = 0.07
_TOP_K = 15
_RKS_COLS = 50

_NC = 2
_NS = 16
_NW = _NC * _NS
_ROWS_PER_W = 64
_PAD_ROWS = _NW * _ROWS_PER_W


def _sc_gather_body(indices_hbm, rks_hbm, feat_hbm, out_hbm,
                    idx_v, blk_v, nb_v, l_v, neigh_v, rows_v, shared_v, sem):
    c = lax.axis_index("c")
    s = lax.axis_index("s")
    wid = s * _NC + c
    base = wid * _ROWS_PER_W
    lanes = lax.iota(jnp.int32, 16)
    pltpu.sync_copy(indices_hbm.at[pl.ds(c * _ROWS_PER_W, _ROWS_PER_W)],
                    idx_v)
    ch0 = (s // 4) * 16
    sel = s - (s // 4) * 4
    chunk = idx_v[pl.ds(ch0, 16)]
    ms = []
    handles = []
    for t in range(4):
        v = jnp.where(sel == 0, chunk[t],
                      jnp.where(sel == 1, chunk[4 + t],
                                jnp.where(sel == 2, chunk[8 + t],
                                          chunk[12 + t])))
        row8 = (v // 8) * 8
        ms.append(v - row8)
        handles.append(pltpu.async_copy(
            rks_hbm.at[pl.ds(row8, 8)],
            blk_v.at[pl.ds(t * 8, 8), :], sem))
    for h in handles:
        h.wait()
    for t in range(4):
        nb_v[pl.ds(t * 16, 16)] = blk_v[t * 8 + ms[t], pl.ds(0, 16)]
    pltpu.sync_copy(nb_v, shared_v.at[pl.ds(s * 64, 64)])
    plsc.subcore_barrier()
    pltpu.sync_copy(shared_v, l_v)
    kvec = jnp.zeros((16,), jnp.int32) + s
    for ch in range(_ROWS_PER_W // 16):
        acc = jnp.zeros((16,), jnp.int32)
        for lane in range(16):
            j = ch * 16 + lane
            row16 = l_v[pl.ds(j * 16, 16)]
            acc = jnp.where(lanes == lane, jnp.take(row16, kvec), acc)
        neigh_v[pl.ds(ch * 16, 16)] = acc
    pltpu.async_copy(feat_hbm.at[neigh_v], rows_v, sem).wait()
    pltpu.sync_copy(rows_v, out_hbm.at[pl.ds(base, _ROWS_PER_W)])


def _sc_gather(indices, saved_rks, saved_features):
    mesh = plsc.VectorSubcoreMesh(core_axis_name="c", subcore_axis_name="s")
    return pl.kernel(
        _sc_gather_body,
        out_type=jax.ShapeDtypeStruct((_PAD_ROWS, 128), jnp.float32),
        mesh=mesh,
        compiler_params=pltpu.CompilerParams(use_tc_tiling_on_sc=True),
        scratch_types=[
            pltpu.VMEM((_ROWS_PER_W,), jnp.int32),
            pltpu.VMEM((32, _RKS_COLS), jnp.int32),
            pltpu.VMEM((_ROWS_PER_W,), jnp.int32),
            pltpu.VMEM((_ROWS_PER_W * 16,), jnp.int32),
            pltpu.VMEM((_ROWS_PER_W,), jnp.int32),
            pltpu.VMEM((_ROWS_PER_W, 128), jnp.float32),
            pltpu.VMEM_SHARED((_ROWS_PER_W * 16,), jnp.int32),
            pltpu.SemaphoreType.DMA,
        ],
    )(indices, saved_rks, saved_features)


def _sim_from(d2):
    d = jnp.sqrt(jnp.maximum(d2, 0.0))
    return (1.0 + 1.0 / (1.0 + d)) / _TEMP


def _dotT(x, y):
    return lax.dot_general(x, y, (((1,), (1,)), ((), ())),
                           preferred_element_type=jnp.float32,
                           precision=lax.Precision.HIGHEST)


def _tc_loss_body(a_ref, g_ref, out_ref):
    A = a_ref[...]
    a2 = jnp.sum(A * A, axis=1)

    paa = _dotT(A, A)
    d2aa = a2[:, None] + a2[None, :] - 2.0 * paa
    orig = _sim_from(d2aa)

    S = jnp.zeros((256, 128), jnp.float32)
    ST = jnp.zeros((128, 256), jnp.float32)
    for k in range(_TOP_K):
        Gk = g_ref[pl.ds(k * 128, 128), :]
        g2 = jnp.sum(Gk * Gk, axis=1)
        p = _dotT(A, Gk)
        S = S + _sim_from(a2[:, None] + g2[None, :] - 2.0 * p)
        pT = _dotT(Gk, A)
        ST = ST + _sim_from(g2[:, None] + a2[None, :] - 2.0 * pT)

    Sn = S * (1.0 / _TOP_K)
    SnT = ST * (1.0 / _TOP_K)
    acc = jnp.concatenate([Sn, Sn], axis=1)
    acc_sym = jnp.concatenate([SnT, SnT], axis=0)

    adc = jnp.sqrt(acc * acc + acc_sym * acc_sym + orig * orig)

    rowmax = jnp.max(adc, axis=1)[:, None]
    logits = adc - rowmax
    ii = lax.broadcasted_iota(jnp.int32, (256, 256), 0)
    jj = lax.broadcasted_iota(jnp.int32, (256, 256), 1)
    offdiag = ii != jj
    pos = jnp.abs(ii - jj) == 128
    denom = jnp.sum(jnp.where(offdiag, jnp.exp(logits), 0.0), axis=1)
    posval = jnp.sum(jnp.where(pos, logits, 0.0), axis=1)
    loss_i = jnp.log(denom) - posval
    out_ref[0, 0] = jnp.mean(loss_i)


def _tc_loss(A, G):
    return pl.pallas_call(
        _tc_loss_body,
        out_shape=jax.ShapeDtypeStruct((1, 1), jnp.float32),
        in_specs=[
            pl.BlockSpec(memory_space=pltpu.VMEM),
            pl.BlockSpec(memory_space=pltpu.VMEM),
        ],
        out_specs=pl.BlockSpec(memory_space=pltpu.SMEM),
    )(A, G)


def kernel(features, indices, saved_features, saved_rks):
    A = jnp.concatenate([features[:, 0, :], features[:, 1, :]], axis=0)
    G = _sc_gather(indices, saved_rks, saved_features)
    out = _tc_loss(A, G)
    return out[0, 0]


# speedup vs baseline: 14.5053x; 2.0620x over previous
import functools

import jax
import jax.numpy as jnp
from jax import lax
from jax.experimental import pallas as pl
from jax.experimental.pallas import tpu as pltpu
from jax.experimental.pallas import tpu_sc as plsc

_TEMP = 0.07
_TOP_K = 15
_RKS_COLS = 50

_NC = 2
_NS = 16
_NW = _NC * _NS
_ROWS_PER_W = 64
_PAD_ROWS = _NW * _ROWS_PER_W


def _sc_gather_body(indices_hbm, rks_hbm, feat_hbm, out_hbm,
                    idx_v, blk_v, nb_v, l_v, neigh_v, rows_v, shared_v, sem):
    c = lax.axis_index("c")
    s = lax.axis_index("s")
    wid = s * _NC + c
    base = wid * _ROWS_PER_W
    lanes = lax.iota(jnp.int32, 16)
    pltpu.sync_copy(indices_hbm.at[pl.ds(c * _ROWS_PER_W, _ROWS_PER_W)],
                    idx_v)
    ch0 = (s // 4) * 16
    sel = s - (s // 4) * 4
    chunk = idx_v[pl.ds(ch0, 16)]
    vs = []
    handles = []
    for t in range(4):
        v = jnp.where(sel == 0, chunk[t],
                      jnp.where(sel == 1, chunk[4 + t],
                                jnp.where(sel == 2, chunk[8 + t],
                                          chunk[12 + t])))
        vs.append(v)
        col128 = (v // 128) * 128
        for h2 in range(2):
            handles.append(pltpu.async_copy(
                rks_hbm.at[pl.ds(h2 * 8, 8), pl.ds(col128, 128)],
                blk_v.at[pl.ds((t * 2 + h2) * 8, 8), :], sem))
    for h in handles:
        h.wait()
    for t in range(4):
        l = vs[t] - (vs[t] // 128) * 128
        loff = (l // 16) * 16
        lvec = jnp.zeros((16,), jnp.int32) + (l - loff)
        acc = jnp.zeros((16,), jnp.int32)
        for kk in range(16):
            row = (t * 2 + kk // 8) * 8 + kk % 8
            c16 = blk_v[row, pl.ds(loff, 16)]
            acc = jnp.where(lanes == kk, jnp.take(c16, lvec), acc)
        nb_v[pl.ds(t * 16, 16)] = acc
    pltpu.sync_copy(nb_v, shared_v.at[pl.ds(s * 64, 64)])
    plsc.subcore_barrier()
    pltpu.sync_copy(shared_v, l_v)
    kvec = jnp.zeros((16,), jnp.int32) + s
    for ch in range(_ROWS_PER_W // 16):
        acc = jnp.zeros((16,), jnp.int32)
        for lane in range(16):
            j = ch * 16 + lane
            row16 = l_v[pl.ds(j * 16, 16)]
            acc = jnp.where(lanes == lane, jnp.take(row16, kvec), acc)
        neigh_v[pl.ds(ch * 16, 16)] = acc
    pltpu.async_copy(feat_hbm.at[neigh_v], rows_v, sem).wait()
    pltpu.sync_copy(rows_v, out_hbm.at[pl.ds(base, _ROWS_PER_W)])


def _sc_gather(indices, saved_rks, saved_features):
    mesh = plsc.VectorSubcoreMesh(core_axis_name="c", subcore_axis_name="s")
    return pl.kernel(
        _sc_gather_body,
        out_type=jax.ShapeDtypeStruct((_PAD_ROWS, 128), jnp.float32),
        mesh=mesh,
        compiler_params=pltpu.CompilerParams(use_tc_tiling_on_sc=True),
        scratch_types=[
            pltpu.VMEM((_ROWS_PER_W,), jnp.int32),
            pltpu.VMEM((64, 128), jnp.int32),
            pltpu.VMEM((_ROWS_PER_W,), jnp.int32),
            pltpu.VMEM((_ROWS_PER_W * 16,), jnp.int32),
            pltpu.VMEM((_ROWS_PER_W,), jnp.int32),
            pltpu.VMEM((_ROWS_PER_W, 128), jnp.float32),
            pltpu.VMEM_SHARED((_ROWS_PER_W * 16,), jnp.int32),
            pltpu.SemaphoreType.DMA,
        ],
    )(indices, saved_rks, saved_features)


def _sim_from(d2):
    d = jnp.sqrt(jnp.maximum(d2, 0.0))
    return (1.0 + 1.0 / (1.0 + d)) / _TEMP


def _dotT(x, y):
    return lax.dot_general(x, y, (((1,), (1,)), ((), ())),
                           preferred_element_type=jnp.float32,
                           precision=lax.Precision.HIGHEST)


def _tc_loss_body(a_ref, g_ref, out_ref):
    A = a_ref[...]
    a2 = jnp.sum(A * A, axis=1)

    paa = _dotT(A, A)
    d2aa = a2[:, None] + a2[None, :] - 2.0 * paa
    orig = _sim_from(d2aa)

    S = jnp.zeros((256, 128), jnp.float32)
    ST = jnp.zeros((128, 256), jnp.float32)
    for k in range(_TOP_K):
        Gk = g_ref[pl.ds(k * 128, 128), :]
        g2 = jnp.sum(Gk * Gk, axis=1)
        p = _dotT(A, Gk)
        S = S + _sim_from(a2[:, None] + g2[None, :] - 2.0 * p)
        pT = _dotT(Gk, A)
        ST = ST + _sim_from(g2[:, None] + a2[None, :] - 2.0 * pT)

    Sn = S * (1.0 / _TOP_K)
    SnT = ST * (1.0 / _TOP_K)
    acc = jnp.concatenate([Sn, Sn], axis=1)
    acc_sym = jnp.concatenate([SnT, SnT], axis=0)

    adc = jnp.sqrt(acc * acc + acc_sym * acc_sym + orig * orig)

    rowmax = jnp.max(adc, axis=1)[:, None]
    logits = adc - rowmax
    ii = lax.broadcasted_iota(jnp.int32, (256, 256), 0)
    jj = lax.broadcasted_iota(jnp.int32, (256, 256), 1)
    offdiag = ii != jj
    pos = jnp.abs(ii - jj) == 128
    denom = jnp.sum(jnp.where(offdiag, jnp.exp(logits), 0.0), axis=1)
    posval = jnp.sum(jnp.where(pos, logits, 0.0), axis=1)
    loss_i = jnp.log(denom) - posval
    out_ref[0, 0] = jnp.mean(loss_i)


def _tc_loss(A, G):
    return pl.pallas_call(
        _tc_loss_body,
        out_shape=jax.ShapeDtypeStruct((1, 1), jnp.float32),
        in_specs=[
            pl.BlockSpec(memory_space=pltpu.VMEM),
            pl.BlockSpec(memory_space=pltpu.VMEM),
        ],
        out_specs=pl.BlockSpec(memory_space=pltpu.SMEM),
    )(A, G)


def kernel(features, indices, saved_features, saved_rks):
    A = jnp.concatenate([features[:, 0, :], features[:, 1, :]], axis=0)
    G = _sc_gather(indices, saved_rks.T, saved_features)
    out = _tc_loss(A, G)
    return out[0, 0]


# speedup vs baseline: 15.4695x; 1.0665x over previous
import functools

import jax
import jax.numpy as jnp
from jax import lax
from jax.experimental import pallas as pl
from jax.experimental.pallas import tpu as pltpu
from jax.experimental.pallas import tpu_sc as plsc

_TEMP = 0.07
_TOP_K = 15
_RKS_COLS = 50

_NC = 2
_NS = 16
_NW = _NC * _NS
_ROWS_PER_W = 64
_PAD_ROWS = _NW * _ROWS_PER_W


def _sc_gather_body(indices_hbm, rks_hbm, feat_hbm, out_hbm,
                    idx_v, blk_v, nb_v, l_v, neigh_v, rows_v, shared_v, sem):
    c = lax.axis_index("c")
    s = lax.axis_index("s")
    wid = s * _NC + c
    base = wid * _ROWS_PER_W
    lanes = lax.iota(jnp.int32, 16)
    pltpu.sync_copy(indices_hbm.at[pl.ds(c * _ROWS_PER_W, _ROWS_PER_W)],
                    idx_v)
    ch0 = (s // 4) * 16
    sel = s - (s // 4) * 4
    chunk = idx_v[pl.ds(ch0, 16)]
    vs = []
    handles = []
    for t in range(4):
        v = jnp.where(sel == 0, chunk[t],
                      jnp.where(sel == 1, chunk[4 + t],
                                jnp.where(sel == 2, chunk[8 + t],
                                          chunk[12 + t])))
        vs.append(v)
        col128 = (v // 128) * 128
        for h2 in range(2):
            handles.append(pltpu.async_copy(
                rks_hbm.at[pl.ds(h2 * 8, 8), pl.ds(col128, 128)],
                blk_v.at[pl.ds((t * 2 + h2) * 8, 8), :], sem))
    for h in handles:
        h.wait()
    for t in range(4):
        l = vs[t] - (vs[t] // 128) * 128
        loff = (l // 16) * 16
        lvec = jnp.zeros((16,), jnp.int32) + (l - loff)
        acc = jnp.zeros((16,), jnp.int32)
        for kk in range(16):
            row = (t * 2 + kk // 8) * 8 + kk % 8
            c16 = blk_v[row, pl.ds(loff, 16)]
            acc = jnp.where(lanes == kk, jnp.take(c16, lvec), acc)
        nb_v[pl.ds(t * 16, 16)] = acc
    pltpu.sync_copy(nb_v, shared_v.at[pl.ds(s * 64, 64)])
    plsc.subcore_barrier()
    pltpu.sync_copy(shared_v, l_v)
    kvec = jnp.zeros((16,), jnp.int32) + s
    for ch in range(_ROWS_PER_W // 16):
        acc = jnp.zeros((16,), jnp.int32)
        for lane in range(16):
            j = ch * 16 + lane
            row16 = l_v[pl.ds(j * 16, 16)]
            acc = jnp.where(lanes == lane, jnp.take(row16, kvec), acc)
        neigh_v[pl.ds(ch * 16, 16)] = acc
    pltpu.async_copy(feat_hbm.at[neigh_v], rows_v, sem).wait()
    pltpu.sync_copy(rows_v, out_hbm.at[pl.ds(base, _ROWS_PER_W)])


def _sc_gather(indices, saved_rks, saved_features):
    mesh = plsc.VectorSubcoreMesh(core_axis_name="c", subcore_axis_name="s")
    return pl.kernel(
        _sc_gather_body,
        out_type=jax.ShapeDtypeStruct((_PAD_ROWS, 128), jnp.float32),
        mesh=mesh,
        compiler_params=pltpu.CompilerParams(use_tc_tiling_on_sc=True),
        scratch_types=[
            pltpu.VMEM((_ROWS_PER_W,), jnp.int32),
            pltpu.VMEM((64, 128), jnp.int32),
            pltpu.VMEM((_ROWS_PER_W,), jnp.int32),
            pltpu.VMEM((_ROWS_PER_W * 16,), jnp.int32),
            pltpu.VMEM((_ROWS_PER_W,), jnp.int32),
            pltpu.VMEM((_ROWS_PER_W, 128), jnp.float32),
            pltpu.VMEM_SHARED((_ROWS_PER_W * 16,), jnp.int32),
            pltpu.SemaphoreType.DMA,
        ],
    )(indices, saved_rks, saved_features)


def _sim_from(d2):
    d = jnp.sqrt(jnp.maximum(d2, 0.0))
    return (1.0 + 1.0 / (1.0 + d)) / _TEMP


def _dotT(x, y):
    return lax.dot_general(x, y, (((1,), (1,)), ((), ())),
                           preferred_element_type=jnp.float32,
                           precision=lax.Precision.DEFAULT)


def _tc_loss_body(a_ref, g_ref, out_ref):
    A = a_ref[...]
    a2 = jnp.sum(A * A, axis=1)

    paa = _dotT(A, A)
    d2aa = a2[:, None] + a2[None, :] - 2.0 * paa
    orig = _sim_from(d2aa)

    S = jnp.zeros((256, 128), jnp.float32)
    ST = jnp.zeros((128, 256), jnp.float32)
    for k in range(_TOP_K):
        Gk = g_ref[pl.ds(k * 128, 128), :]
        g2 = jnp.sum(Gk * Gk, axis=1)
        p = _dotT(A, Gk)
        S = S + _sim_from(a2[:, None] + g2[None, :] - 2.0 * p)
        pT = _dotT(Gk, A)
        ST = ST + _sim_from(g2[:, None] + a2[None, :] - 2.0 * pT)

    Sn = S * (1.0 / _TOP_K)
    SnT = ST * (1.0 / _TOP_K)
    acc = jnp.concatenate([Sn, Sn], axis=1)
    acc_sym = jnp.concatenate([SnT, SnT], axis=0)

    adc = jnp.sqrt(acc * acc + acc_sym * acc_sym + orig * orig)

    rowmax = jnp.max(adc, axis=1)[:, None]
    logits = adc - rowmax
    ii = lax.broadcasted_iota(jnp.int32, (256, 256), 0)
    jj = lax.broadcasted_iota(jnp.int32, (256, 256), 1)
    offdiag = ii != jj
    pos = jnp.abs(ii - jj) == 128
    denom = jnp.sum(jnp.where(offdiag, jnp.exp(logits), 0.0), axis=1)
    posval = jnp.sum(jnp.where(pos, logits, 0.0), axis=1)
    loss_i = jnp.log(denom) - posval
    out_ref[0, 0] = jnp.mean(loss_i)


def _tc_loss(A, G):
    return pl.pallas_call(
        _tc_loss_body,
        out_shape=jax.ShapeDtypeStruct((1, 1), jnp.float32),
        in_specs=[
            pl.BlockSpec(memory_space=pltpu.VMEM),
            pl.BlockSpec(memory_space=pltpu.VMEM),
        ],
        out_specs=pl.BlockSpec(memory_space=pltpu.SMEM),
    )(A, G)


def kernel(features, indices, saved_features, saved_rks):
    A = jnp.concatenate([features[:, 0, :], features[:, 1, :]], axis=0)
    G = _sc_gather(indices, saved_rks.T, saved_features)
    out = _tc_loss(A, G)
    return out[0, 0]
